# Initial kernel scaffold; baseline (speedup 1.0000x reference)
import jax
import jax.numpy as jnp
from jax.experimental import pallas as pl


def kernel(x, conv_w, conv_b):
    raise NotImplementedError("write your pallas kernel here")


# speedup vs baseline: 1.7340x; 1.7340x over previous
import jax
import jax.numpy as jnp
from jax.experimental import pallas as pl
from jax.experimental.pallas import tpu as pltpu

_VMEM_LIMIT_BYTES = 48 * 1024 * 1024


def _conv_kernel(ph_ref, w_ref, b_ref, o_ref, patch_ref):
    C = w_ref.shape[1] // 9
    M = patch_ref.shape[1]
    Ho = M - o_ref.shape[2]
    Wp = M // Ho
    Wo = Wp - 1

    for ky in range(3):
        for kx in range(3):
            tap = ky * 3 + kx
            p = (ky % 2) * 2 + (kx % 2)
            off = (ky // 2) * Wp + (kx // 2)
            patch_ref[tap * C:(tap + 1) * C, :] = (
                ph_ref[0, p * C:(p + 1) * C, off:off + M])

    acc = jnp.dot(w_ref[...], patch_ref[...],
                  preferred_element_type=jnp.float32)
    acc = acc + b_ref[...]
    for oy in range(Ho):
        o_ref[0, :, oy * Wo:(oy + 1) * Wo] = acc[:, oy * Wp:oy * Wp + Wo]


def kernel(x, conv_w, conv_b):
    B, C, H, W = x.shape
    OC = conv_w.shape[0]
    Ho, Wo = H // 2, W // 2
    Wp = Wo + 1
    Np = (Ho + 1) * Wp
    M = Ho * Wp
    NPAD = (M + Wp + 1 + 127) // 128 * 128

    xp = jnp.pad(x, ((0, 0), (0, 0), (1, 1), (1, 1)))
    ph = xp.reshape(B, C, Ho + 1, 2, Wp, 2)
    ph = ph.transpose(0, 3, 5, 1, 2, 4).reshape(B, 4 * C, Np)
    ph = jnp.pad(ph, ((0, 0), (0, 0), (0, NPAD - Np))).astype(jnp.bfloat16)

    w2 = conv_w.transpose(0, 2, 3, 1).reshape(OC, 9 * C).astype(jnp.bfloat16)
    b2 = conv_b.reshape(OC, 1).astype(jnp.float32)

    out = pl.pallas_call(
        _conv_kernel,
        out_shape=jax.ShapeDtypeStruct((B, OC, Ho * Wo), jnp.float32),
        grid=(B,),
        in_specs=[
            pl.BlockSpec((1, 4 * C, NPAD), lambda i: (i, 0, 0)),
            pl.BlockSpec((OC, 9 * C), lambda i: (0, 0)),
            pl.BlockSpec((OC, 1), lambda i: (0, 0)),
        ],
        out_specs=pl.BlockSpec((1, OC, Ho * Wo), lambda i: (i, 0, 0)),
        scratch_shapes=[pltpu.VMEM((9 * C, M), jnp.bfloat16)],
        compiler_params=pltpu.CompilerParams(
            dimension_semantics=("parallel",),
            vmem_limit_bytes=_VMEM_LIMIT_BYTES),
    )(ph, w2, b2)
    return out.reshape(B, OC, Ho, Wo)
